# baseline (speedup 1.0000x reference)
import dataclasses
import functools

import jax
import jax.numpy as jnp
from jax import lax
from jax.experimental import pallas as pl
from jax.experimental.pallas import tpu as pltpu
from jax.experimental.pallas import tpu_sc as plsc

D = 128
LANES = 16
NUM_CORES = 2
NUM_SUBCORES = 16
NW = NUM_CORES * NUM_SUBCORES


NBUF = 2


def _dot_kernel(E, W, ufeat_hbm, ifeat_hbm, src_hbm, dst_hbm, out_hbm,
                idx_u, idx_v, u_a, v_a, u_b, v_b, red, out_all,
                sem_ua, sem_va, sem_ub, sem_vb):
    per_w = E // NW
    n_chunks = per_w // W
    wid = lax.axis_index("s") * NUM_CORES + lax.axis_index("c")
    base_w = wid * per_w
    col0 = lax.iota(jnp.int32, LANES) * LANES

    pltpu.sync_copy(src_hbm.at[pl.ds(base_w, per_w)], idx_u)
    pltpu.sync_copy(dst_hbm.at[pl.ds(base_w, per_w)], idx_v)

    bufs = [(u_a, v_a, sem_ua, sem_va), (u_b, v_b, sem_ub, sem_vb)]

    def gather(c, buf):
        u_buf, v_buf, sem_u, sem_v = buf
        pltpu.make_async_copy(
            ufeat_hbm.at[idx_u.at[pl.ds(c * W, W)]], u_buf, sem_u).start()
        pltpu.make_async_copy(
            ifeat_hbm.at[idx_v.at[pl.ds(c * W, W)]], v_buf, sem_v).start()

    def wait(c, buf):
        u_buf, v_buf, sem_u, sem_v = buf
        pltpu.make_async_copy(
            ufeat_hbm.at[idx_u.at[pl.ds(c * W, W)]], u_buf, sem_u).wait()
        pltpu.make_async_copy(
            ifeat_hbm.at[idx_v.at[pl.ds(c * W, W)]], v_buf, sem_v).wait()

    hi_mask = jnp.full((LANES,), 0xFFFF0000, dtype=jnp.uint32)

    def compute(c, buf):
        u_buf, v_buf = buf[0], buf[1]
        @plsc.parallel_loop(0, W // LANES)
        def _(g):
            rbase = g * LANES * LANES
            for e in range(LANES):
                row = g * LANES + e
                prods = [u_buf[row, pl.ds(k * 2 * LANES, 2 * LANES)]
                         * v_buf[row, pl.ds(k * 2 * LANES, 2 * LANES)]
                         for k in range(D // (2 * LANES))]
                while len(prods) > 1:
                    prods = [prods[i] + prods[i + 1]
                             for i in range(0, len(prods) - 1, 2)] \
                        + ([prods[-1]] if len(prods) % 2 else [])
                w = plsc.bitcast(prods[0], jnp.uint32)
                hi = plsc.bitcast(w & hi_mask, jnp.float32)
                lo = plsc.bitcast(w << 16, jnp.float32)
                red[pl.ds(rbase + e * LANES, LANES)] = lo + hi
            cols = [plsc.load_gather(red, [rbase + col0 + j])
                    for j in range(LANES)]
            while len(cols) > 1:
                cols = [cols[i] + cols[i + 1]
                        for i in range(0, len(cols) - 1, 2)] \
                    + ([cols[-1]] if len(cols) % 2 else [])
            out_all[pl.ds(c * W + g * LANES, LANES)] = cols[0]

    r = n_chunks - (n_chunks // NBUF) * NBUF
    if r == 0:
        r = NBUF
    body_hi = n_chunks - r

    for i in range(NBUF):
        gather(i, bufs[i])

    @pl.loop(0, body_hi, step=NBUF)
    def _(c):
        for i in range(NBUF):
            wait(c + i, bufs[i])
            compute(c + i, bufs[i])
            nxt = c + i + NBUF

            @pl.when(nxt < n_chunks)
            def _():
                gather(nxt, bufs[i])

    for i in range(r):
        wait(body_hi + i, bufs[i])
        compute(body_hi + i, bufs[i])

    pltpu.sync_copy(out_all, out_hbm.at[pl.ds(base_w, per_w)])


def _build_sc_call(E, W):
    per_w = E // NW
    mesh = plsc.VectorSubcoreMesh(core_axis_name="c", subcore_axis_name="s")
    cp = pltpu.CompilerParams()
    if "needs_layout_passes" in pltpu.CompilerParams.__dataclass_fields__:
        cp = dataclasses.replace(cp, needs_layout_passes=False)
    if "use_tc_tiling_on_sc" in pltpu.CompilerParams.__dataclass_fields__:
        cp = dataclasses.replace(cp, use_tc_tiling_on_sc=False)
    return pl.kernel(
        functools.partial(_dot_kernel, E, W),
        out_type=jax.ShapeDtypeStruct((E,), jnp.float32),
        mesh=mesh,
        scratch_types=[
            pltpu.VMEM((per_w,), jnp.int32),
            pltpu.VMEM((per_w,), jnp.int32),
        ] + [pltpu.VMEM((W, D), jnp.bfloat16) for _ in range(2 * NBUF)] + [
            pltpu.VMEM((W * LANES,), jnp.float32),
            pltpu.VMEM((per_w,), jnp.float32),
        ] + [pltpu.SemaphoreType.DMA for _ in range(2 * NBUF)],
        compiler_params=cp,
    )


@jax.jit
def kernel(ufeat, ifeat, edge_index):
    E = edge_index.shape[1]
    src = edge_index[0].astype(jnp.int32)
    dst = edge_index[1].astype(jnp.int32)

    sr = _build_sc_call(E, 80)(ufeat.astype(jnp.bfloat16),
                               ifeat.astype(jnp.bfloat16), src, dst)
    return sr.reshape(E, 1)


# speedup vs baseline: 1.0286x; 1.0286x over previous
import dataclasses
import functools

import jax
import jax.numpy as jnp
from jax import lax
from jax.experimental import pallas as pl
from jax.experimental.pallas import tpu as pltpu
from jax.experimental.pallas import tpu_sc as plsc

D = 128
LANES = 16
NUM_CORES = 2
NUM_SUBCORES = 16
NW = NUM_CORES * NUM_SUBCORES


NBUF = 2


def _dot_kernel(E, W, ufeat_hbm, ifeat_hbm, src_hbm, dst_hbm, out_hbm,
                idx_u, idx_v, u_a, v_a, u_b, v_b, red, out_all,
                sem_ua, sem_va, sem_ub, sem_vb):
    per_w = E // NW
    n_chunks = per_w // W
    wid = lax.axis_index("s") * NUM_CORES + lax.axis_index("c")
    base_w = wid * per_w
    col0 = lax.iota(jnp.int32, LANES) * LANES

    pltpu.sync_copy(src_hbm.at[pl.ds(base_w, per_w)], idx_u)
    pltpu.sync_copy(dst_hbm.at[pl.ds(base_w, per_w)], idx_v)

    bufs = [(u_a, v_a, sem_ua, sem_va), (u_b, v_b, sem_ub, sem_vb)]

    def gather(c, buf):
        u_buf, v_buf, sem_u, sem_v = buf
        pltpu.make_async_copy(
            ufeat_hbm.at[idx_u.at[pl.ds(c * W, W)]], u_buf, sem_u).start()
        pltpu.make_async_copy(
            ifeat_hbm.at[idx_v.at[pl.ds(c * W, W)]], v_buf, sem_v).start()

    def wait(c, buf):
        u_buf, v_buf, sem_u, sem_v = buf
        pltpu.make_async_copy(
            ufeat_hbm.at[idx_u.at[pl.ds(c * W, W)]], u_buf, sem_u).wait()
        pltpu.make_async_copy(
            ifeat_hbm.at[idx_v.at[pl.ds(c * W, W)]], v_buf, sem_v).wait()

    hi_mask = jnp.full((LANES,), 0xFFFF0000, dtype=jnp.uint32)

    def compute(c, buf):
        u_buf, v_buf = buf[0], buf[1]
        @pl.loop(0, W // LANES)
        def _(g):
            rbase = g * LANES * LANES
            for e in range(LANES):
                row = g * LANES + e
                prods = [u_buf[row, pl.ds(k * 2 * LANES, 2 * LANES)]
                         * v_buf[row, pl.ds(k * 2 * LANES, 2 * LANES)]
                         for k in range(D // (2 * LANES))]
                while len(prods) > 1:
                    prods = [prods[i] + prods[i + 1]
                             for i in range(0, len(prods) - 1, 2)] \
                        + ([prods[-1]] if len(prods) % 2 else [])
                w = plsc.bitcast(prods[0], jnp.uint32)
                hi = plsc.bitcast(w & hi_mask, jnp.float32)
                lo = plsc.bitcast(w << 16, jnp.float32)
                red[pl.ds(rbase + e * LANES, LANES)] = lo + hi
            cols = [plsc.load_gather(red, [rbase + col0 + j])
                    for j in range(LANES)]
            while len(cols) > 1:
                cols = [cols[i] + cols[i + 1]
                        for i in range(0, len(cols) - 1, 2)] \
                    + ([cols[-1]] if len(cols) % 2 else [])
            out_all[pl.ds(c * W + g * LANES, LANES)] = cols[0]

    r = n_chunks - (n_chunks // NBUF) * NBUF
    if r == 0:
        r = NBUF
    body_hi = n_chunks - r

    for i in range(NBUF):
        gather(i, bufs[i])

    @pl.loop(0, body_hi, step=NBUF)
    def _(c):
        for i in range(NBUF):
            wait(c + i, bufs[i])
            compute(c + i, bufs[i])
            nxt = c + i + NBUF

            @pl.when(nxt < n_chunks)
            def _():
                gather(nxt, bufs[i])

    for i in range(r):
        wait(body_hi + i, bufs[i])
        compute(body_hi + i, bufs[i])

    pltpu.sync_copy(out_all, out_hbm.at[pl.ds(base_w, per_w)])


def _build_sc_call(E, W):
    per_w = E // NW
    mesh = plsc.VectorSubcoreMesh(core_axis_name="c", subcore_axis_name="s")
    cp = pltpu.CompilerParams()
    if "needs_layout_passes" in pltpu.CompilerParams.__dataclass_fields__:
        cp = dataclasses.replace(cp, needs_layout_passes=False)
    if "use_tc_tiling_on_sc" in pltpu.CompilerParams.__dataclass_fields__:
        cp = dataclasses.replace(cp, use_tc_tiling_on_sc=False)
    return pl.kernel(
        functools.partial(_dot_kernel, E, W),
        out_type=jax.ShapeDtypeStruct((E,), jnp.float32),
        mesh=mesh,
        scratch_types=[
            pltpu.VMEM((per_w,), jnp.int32),
            pltpu.VMEM((per_w,), jnp.int32),
        ] + [pltpu.VMEM((W, D), jnp.bfloat16) for _ in range(2 * NBUF)] + [
            pltpu.VMEM((W * LANES,), jnp.float32),
            pltpu.VMEM((per_w,), jnp.float32),
        ] + [pltpu.SemaphoreType.DMA for _ in range(2 * NBUF)],
        compiler_params=cp,
    )


@jax.jit
def kernel(ufeat, ifeat, edge_index):
    E = edge_index.shape[1]
    src = edge_index[0].astype(jnp.int32)
    dst = edge_index[1].astype(jnp.int32)

    sr = _build_sc_call(E, 80)(ufeat.astype(jnp.bfloat16),
                               ifeat.astype(jnp.bfloat16), src, dst)
    return sr.reshape(E, 1)
